# Initial kernel scaffold; baseline (speedup 1.0000x reference)
import jax
import jax.numpy as jnp
from jax.experimental import pallas as pl


def kernel(features, node, neighbours, neigh_weights):
    raise NotImplementedError("write your pallas kernel here")


# speedup vs baseline: 1.3447x; 1.3447x over previous
import functools

import jax
import jax.numpy as jnp
from jax import lax
from jax.experimental import pallas as pl
from jax.experimental.pallas import tpu as pltpu
from jax.experimental.pallas import tpu_sc as plsc

D = 512
B = 8192
K = 17
LANE = 16

NC, NS = 2, 16
NW = NC * NS
EPW = B // NW
CHUNK = 8
NCHUNK = EPW // CHUNK
ROWS = CHUNK * K
COLV = D // LANE

_mesh = plsc.VectorSubcoreMesh(
    core_axis_name="c", subcore_axis_name="s", num_cores=NC, num_subcores=NS
)


@functools.partial(
    pl.kernel,
    out_type=jax.ShapeDtypeStruct((B, D), jnp.float32),
    mesh=_mesh,
    scratch_types=[
        pltpu.VMEM((ROWS,), jnp.int32),
        pltpu.VMEM((ROWS, D), jnp.float32),
        pltpu.VMEM((CHUNK, D), jnp.float32),
        pltpu.SemaphoreType.DMA,
    ],
)
def _gather_sum(feat_hbm, idx_hbm, out_hbm, idx_v, rows_v, acc_v, sem):
    wid = lax.axis_index("s") * NC + lax.axis_index("c")

    def chunk_body(ci, carry):
        ebase = wid * EPW + ci * CHUNK
        pltpu.sync_copy(idx_hbm.at[pl.ds(ebase * K, ROWS)], idx_v)
        pltpu.async_copy(feat_hbm.at[idx_v], rows_v, sem).wait()

        def col_body(cv, c2):
            sl = pl.ds(cv * LANE, LANE)
            for j in range(CHUNK):
                acc = rows_v[j * K, sl]
                for r in range(1, K):
                    acc = acc + rows_v[j * K + r, sl]
                acc_v[j, sl] = acc
            return c2

        lax.fori_loop(0, COLV, col_body, 0)
        pltpu.sync_copy(acc_v, out_hbm.at[pl.ds(ebase, CHUNK)])
        return carry

    lax.fori_loop(0, NCHUNK, chunk_body, 0)


BM = 1024


def _mm_body(x_ref, w_ref, o_ref):
    y = jnp.dot(x_ref[...], w_ref[...], preferred_element_type=jnp.float32)
    o_ref[...] = jnp.maximum(y * (1.0 / K), 0.0)


def _matmul_relu(x, w):
    return pl.pallas_call(
        _mm_body,
        grid=(B // BM,),
        in_specs=[
            pl.BlockSpec((BM, D), lambda i: (i, 0)),
            pl.BlockSpec((D, D), lambda i: (0, 0)),
        ],
        out_specs=pl.BlockSpec((BM, D), lambda i: (i, 0)),
        out_shape=jax.ShapeDtypeStruct((B, D), jnp.float32),
    )(x, w)


def kernel(features, node, neighbours, neigh_weights):
    idx = jnp.concatenate([neighbours, node], axis=1).reshape(-1)
    sums = _gather_sum(features, idx)
    return _matmul_relu(sums, neigh_weights)
